# baseline (speedup 1.0000x reference)
import functools

import jax
import jax.numpy as jnp
from jax import lax
from jax.experimental import pallas as pl
from jax.experimental.pallas import tpu as pltpu
from jax.experimental.pallas import tpu_sc as plsc

LEFT = 8
RIGHT = 8
ROWS = 8
SLACK = 16


def kernel(features, lengths):
    nb, c, t = features.shape
    w = t + LEFT + RIGHT
    half = c // 2
    nch = half // ROWS
    dtype = features.dtype

    left_vals = lax.rev(lax.slice(features, (0, 0, 1), (1, c, 1 + LEFT)),
                        (2,))[0]
    pads = jnp.concatenate([left_vals, jnp.zeros((c, RIGHT), dtype)],
                           axis=-1)

    mesh = plsc.VectorSubcoreMesh(core_axis_name="c", subcore_axis_name="s")

    @functools.partial(
        pl.kernel,
        mesh=mesh,
        compiler_params=pltpu.CompilerParams(use_tc_tiling_on_sc=False,
                                             needs_layout_passes=False),
        out_type=jax.ShapeDtypeStruct((nb, c, w), dtype),
        scratch_types=[
            pltpu.VMEM((16,), jnp.int32),
            pltpu.VMEM((half, 2 * LEFT), dtype),
            pltpu.VMEM((2, ROWS, w + SLACK), dtype),
            pltpu.SemaphoreType.DMA((2,)),
            pltpu.SemaphoreType.DMA((2,)),
            pltpu.SemaphoreType.DMA,
        ],
    )
    def run(feat, lens, pads_in, out, len_v, pads_v, obuf, sem_g, sem_s,
            sem_p):
        def take16(v, idx):
            return lax.gather(
                v, idx[:, None],
                lax.GatherDimensionNumbers(offset_dims=(),
                                           collapsed_slice_dims=(0,),
                                           start_index_map=(0,)), (1,),
                mode=lax.GatherScatterMode.PROMISE_IN_BOUNDS)

        wid = lax.axis_index("s") * 2 + lax.axis_index("c")
        b = wid // 2
        c0 = pl.multiple_of(lax.rem(wid, 2) * half, half)

        pltpu.sync_copy(lens, len_v)
        lane = lax.broadcasted_iota(jnp.int32, (16,), 0)
        l = jnp.max(jnp.where(lane == b, len_v[...], 0))
        low = lane < LEFT
        low_ok = lane < 16

        hp = pltpu.async_copy(pads_in.at[pl.ds(c0, half)], pads_v, sem_p)

        def gather(k, s):
            r0 = pl.multiple_of(c0 + k * ROWS, ROWS)
            return pltpu.async_copy(feat.at[b, pl.ds(r0, ROWS), :],
                                    obuf.at[s, :, pl.ds(LEFT, t)],
                                    sem_g.at[s])

        def scatter(k, s):
            r0 = pl.multiple_of(c0 + k * ROWS, ROWS)
            return pltpu.async_copy(obuf.at[s, :, pl.ds(0, w)],
                                    out.at[b, pl.ds(r0, ROWS), :],
                                    sem_s.at[s])

        a0 = pl.multiple_of(((l - 1) // 16) * 16, 16)
        aoff = (l - 1) - a0
        d0 = pl.multiple_of(((LEFT + l) // 16) * 16, 16)
        doff = (LEFT + l) - d0
        sidx = aoff + 7 - lane
        s_in_a = sidx < 16
        sia = lax.rem(sidx + 16, 16)
        d0m = (lane >= doff) & (lane < doff + RIGHT) & low_ok
        di0 = lax.rem(lane - doff + 16, 16)
        d1m = lane < doff - RIGHT
        di1 = lax.rem(lane + 16 - doff, 16)

        def patch(k, s):
            for r in range(ROWS):
                prow = pads_v[k * ROWS + r, :]
                head = obuf[s, r, pl.ds(0, 16)]
                obuf[s, r, pl.ds(0, 16)] = jnp.where(low, prow, head)
                tail = obuf[s, r, pl.ds(t, 16)]
                obuf[s, r, pl.ds(t, 16)] = jnp.where(low, tail, prow)
                va = obuf[s, r, pl.ds(a0, 16)]
                vb = obuf[s, r, pl.ds(a0 + 16, 16)]
                strip = jnp.where(s_in_a,
                                  take16(va, sia),
                                  take16(vb, sia))
                vd0 = obuf[s, r, pl.ds(d0, 16)]
                obuf[s, r, pl.ds(d0, 16)] = jnp.where(
                    d0m, take16(strip, di0), vd0)
                vd1 = obuf[s, r, pl.ds(d0 + 16, 16)]
                obuf[s, r, pl.ds(d0 + 16, 16)] = jnp.where(
                    d1m, take16(strip, di1), vd1)

        hp.wait()
        hg = [None] * nch
        hs = [None] * nch
        hg[0] = gather(0, 0)
        for k in range(nch):
            s = k % 2
            if k + 1 < nch:
                if k >= 1:
                    hs[k - 1].wait()
                hg[k + 1] = gather(k + 1, 1 - s)
            hg[k].wait()
            patch(k, s)
            hs[k] = scatter(k, s)
        hs[nch - 2].wait()
        hs[nch - 1].wait()

    return run(features, lengths, pads)


# speedup vs baseline: 2.1464x; 2.1464x over previous
import jax
import jax.numpy as jnp
from jax import lax
from jax.experimental import pallas as pl
from jax.experimental.pallas import tpu as pltpu

LEFT = 8
RIGHT = 8
WIN = 272


def _pad_kernel(lengths_ref, left_ref, feat_ref, out_hbm, main_buf, tail_buf,
                sem_m, sem_t):
    nb, nc = pl.num_programs(0), pl.num_programs(1)
    b, cbk = pl.program_id(0), pl.program_id(1)
    i = b * nc + cbk
    n = nb * nc
    slot = lax.rem(i, 2)
    l = lengths_ref[b]
    feat = feat_ref[0]
    cb, t = feat.shape
    w = t + LEFT + RIGHT
    c0 = cbk * cb

    def body_copy(s):
        return pltpu.make_async_copy(
            main_buf.at[s], out_hbm.at[b, pl.ds(c0, cb), pl.ds(0, t)],
            sem_m.at[s])

    def tail_copy(s):
        return pltpu.make_async_copy(
            tail_buf.at[s], out_hbm.at[b, pl.ds(c0, cb), pl.ds(t, 16)],
            sem_t.at[s])

    @pl.when(i >= 2)
    def _():
        body_copy(slot).wait()
        tail_copy(slot).wait()

    left = left_ref[0]
    zeros = jnp.zeros((cb, RIGHT), feat.dtype)
    base = jnp.concatenate([left, feat, zeros], axis=-1)
    main_buf[slot] = base[:, :t]
    tail_buf[slot] = base[:, t:]

    a = pl.multiple_of(jnp.minimum(((l - 9) // 128) * 128, t - 256), 128)
    win = feat_ref[0, :, pl.ds(a, 256)]
    off = (l - 9) - a
    r1 = pltpu.roll(win, (256 - off) % 256, axis=1)
    s8 = r1[:, :8]
    strip = jnp.concatenate([s8[:, 7 - j:8 - j] for j in range(8)], axis=-1)
    ws = pl.multiple_of(jnp.minimum(((l + LEFT) // 128) * 128, w - WIN), 128)
    poff = (l + LEFT) - ws
    strip_pad = jnp.concatenate(
        [strip, jnp.zeros((cb, WIN - 8), feat.dtype)], axis=-1)
    placed = pltpu.roll(strip_pad, poff, axis=1)
    pos = lax.broadcasted_iota(jnp.int32, (cb, WIN), 1)
    mask = (pos >= poff) & (pos < poff + RIGHT)

    @pl.when(ws < w - WIN)
    def _():
        cur = main_buf[slot, :, pl.ds(ws, WIN)]
        main_buf[slot, :, pl.ds(ws, WIN)] = jnp.where(mask, placed, cur)

    @pl.when(ws == w - WIN)
    def _():
        wse = pl.multiple_of(w - WIN, 128)
        cur = main_buf[slot, :, pl.ds(wse, WIN - 16)]
        main_buf[slot, :, pl.ds(wse, WIN - 16)] = jnp.where(
            mask[:, :WIN - 16], placed[:, :WIN - 16], cur)
        cur_t = tail_buf[slot]
        tail_buf[slot] = jnp.where(mask[:, WIN - 16:], placed[:, WIN - 16:],
                                   cur_t)

    body_copy(slot).start()
    tail_copy(slot).start()

    @pl.when(i == n - 1)
    def _():
        body_copy(slot).wait()
        tail_copy(slot).wait()
        other = 1 - slot

        @pl.when(n >= 2)
        def _():
            body_copy(other).wait()
            tail_copy(other).wait()


def kernel(features, lengths):
    b, c, t = features.shape
    cb = 256
    left_src = lax.rev(
        lax.slice(features, (0, 0, 1), (1, c, 1 + LEFT)), (2,)
    )
    return pl.pallas_call(
        _pad_kernel,
        grid=(b, c // cb),
        in_specs=[
            pl.BlockSpec(memory_space=pltpu.SMEM),
            pl.BlockSpec((1, cb, LEFT), lambda i, j: (0, j, 0)),
            pl.BlockSpec((1, cb, t), lambda i, j: (i, j, 0)),
        ],
        out_specs=pl.BlockSpec(memory_space=pl.ANY),
        out_shape=jax.ShapeDtypeStruct((b, c, t + LEFT + RIGHT), features.dtype),
        scratch_shapes=[
            pltpu.VMEM((2, cb, t), features.dtype),
            pltpu.VMEM((2, cb, LEFT + RIGHT), features.dtype),
            pltpu.SemaphoreType.DMA((2,)),
            pltpu.SemaphoreType.DMA((2,)),
        ],
    )(lengths, left_src, features)


# speedup vs baseline: 2.2279x; 1.0380x over previous
import jax
import jax.numpy as jnp
from jax import lax
from jax.experimental import pallas as pl
from jax.experimental.pallas import tpu as pltpu

LEFT = 8
RIGHT = 8
WIN = 272


def _pad_kernel(lengths_ref, left_ref, feat_ref, out_hbm, main_buf, tail_buf,
                sem_m, sem_t):
    nb, nc = pl.num_programs(0), pl.num_programs(1)
    b, cbk = pl.program_id(0), pl.program_id(1)
    i = b * nc + cbk
    n = nb * nc
    slot = lax.rem(i, 2)
    l = lengths_ref[b]
    feat = feat_ref[0]
    cb, t = feat.shape
    w = t + LEFT + RIGHT
    c0 = cbk * cb

    def body_copy(s):
        return pltpu.make_async_copy(
            main_buf.at[s], out_hbm.at[b, pl.ds(c0, cb), pl.ds(0, t)],
            sem_m.at[s])

    def tail_copy(s):
        return pltpu.make_async_copy(
            tail_buf.at[s], out_hbm.at[b, pl.ds(c0, cb), pl.ds(t, 16)],
            sem_t.at[s])

    @pl.when(i >= 2)
    def _():
        body_copy(slot).wait()
        tail_copy(slot).wait()

    left = left_ref[0]
    zeros = jnp.zeros((cb, RIGHT), feat.dtype)
    base = jnp.concatenate([left, feat, zeros], axis=-1)
    main_buf[slot] = base[:, :t]
    tail_buf[slot] = base[:, t:]

    a = pl.multiple_of(jnp.minimum(((l - 9) // 128) * 128, t - 256), 128)
    win = feat_ref[0, :, pl.ds(a, 256)]
    off = (l - 9) - a
    r1 = pltpu.roll(win, (256 - off) % 256, axis=1)
    s8 = r1[:, :8]
    strip = jnp.concatenate([s8[:, 7 - j:8 - j] for j in range(8)], axis=-1)
    ws = pl.multiple_of(jnp.minimum(((l + LEFT) // 128) * 128, w - WIN), 128)
    poff = (l + LEFT) - ws
    strip_pad = jnp.concatenate(
        [strip, jnp.zeros((cb, WIN - 8), feat.dtype)], axis=-1)
    placed = pltpu.roll(strip_pad, poff, axis=1)
    pos = lax.broadcasted_iota(jnp.int32, (cb, WIN), 1)
    mask = (pos >= poff) & (pos < poff + RIGHT)

    @pl.when(ws < w - WIN)
    def _():
        cur = main_buf[slot, :, pl.ds(ws, WIN)]
        main_buf[slot, :, pl.ds(ws, WIN)] = jnp.where(mask, placed, cur)

    @pl.when(ws == w - WIN)
    def _():
        wse = pl.multiple_of(w - WIN, 128)
        cur = main_buf[slot, :, pl.ds(wse, WIN - 16)]
        main_buf[slot, :, pl.ds(wse, WIN - 16)] = jnp.where(
            mask[:, :WIN - 16], placed[:, :WIN - 16], cur)
        cur_t = tail_buf[slot]
        tail_buf[slot] = jnp.where(mask[:, WIN - 16:], placed[:, WIN - 16:],
                                   cur_t)

    body_copy(slot).start()
    tail_copy(slot).start()

    @pl.when(i == n - 1)
    def _():
        body_copy(slot).wait()
        tail_copy(slot).wait()
        other = 1 - slot

        @pl.when(n >= 2)
        def _():
            body_copy(other).wait()
            tail_copy(other).wait()


def kernel(features, lengths):
    b, c, t = features.shape
    cb = 512
    left_src = lax.rev(
        lax.slice(features, (0, 0, 1), (1, c, 1 + LEFT)), (2,)
    )
    return pl.pallas_call(
        _pad_kernel,
        grid=(b, c // cb),
        in_specs=[
            pl.BlockSpec(memory_space=pltpu.SMEM),
            pl.BlockSpec((1, cb, LEFT), lambda i, j: (0, j, 0)),
            pl.BlockSpec((1, cb, t), lambda i, j: (i, j, 0)),
        ],
        out_specs=pl.BlockSpec(memory_space=pl.ANY),
        out_shape=jax.ShapeDtypeStruct((b, c, t + LEFT + RIGHT), features.dtype),
        scratch_shapes=[
            pltpu.VMEM((2, cb, t), features.dtype),
            pltpu.VMEM((2, cb, LEFT + RIGHT), features.dtype),
            pltpu.SemaphoreType.DMA((2,)),
            pltpu.SemaphoreType.DMA((2,)),
        ],
    )(lengths, left_src, features)
